# baseline (speedup 1.0000x reference)
import functools

import jax
import jax.numpy as jnp
from jax import lax
from jax.experimental import pallas as pl
from jax.experimental.pallas import tpu as pltpu
from jax.experimental.pallas import tpu_sc as plsc

N = 10000
EDG = 320000
D = 128

NC, NS = 2, 16
NW = NC * NS
K = 64
EPT = (EDG + NW - 1) // NW
CH = 8 * ((EPT + 8 * K - 1) // (8 * K))
EPW = CH * K
E_PAD = EPW * NW
NP = N + 112
RPT = NP // NS
NPR = NP // D

f32 = jnp.float32
i32 = jnp.int32
_mesh = plsc.VectorSubcoreMesh(core_axis_name="c", subcore_axis_name="s")


@functools.partial(
    pl.kernel,
    out_type=jax.ShapeDtypeStruct((NW * NP,), i32),
    mesh=_mesh,
    compiler_params=pltpu.CompilerParams(needs_layout_passes=False),
    scratch_types=[
        pltpu.VMEM((CH, K), i32),
        pltpu.VMEM((NP,), i32),
    ],
)
def _deg_kernel(dst_hbm, zeros_hbm, out_hbm, dst_v, hist):
    cid = lax.axis_index("c")
    sid = lax.axis_index("s")
    wid = cid * NS + sid
    pltpu.sync_copy(dst_hbm.at[wid], dst_v)
    pltpu.sync_copy(zeros_hbm, hist)

    def body(c, carry):
        for j in range(K // 16):
            dv = dst_v[c, pl.ds(j * 16, 16)]
            cnt, last = plsc.scan_count(dv)
            plsc.addupdate_scatter(hist, [dv], cnt, mask=last)
        return carry

    lax.fori_loop(0, CH, body, 0)
    pltpu.sync_copy(hist, out_hbm.at[pl.ds(wid * NP, NP)])


@functools.partial(
    pl.kernel,
    out_type=jax.ShapeDtypeStruct((NC, NP, D), f32),
    mesh=_mesh,
    scratch_types=[
        pltpu.VMEM((CH // 4, K), i32),
        pltpu.VMEM((CH // 4, K), i32),
        pltpu.VMEM((K, D), f32),
        pltpu.VMEM((K, D), f32),
        pltpu.VMEM((K, D), f32),
        pltpu.VMEM((K, D), f32),
        pltpu.VMEM_SHARED((NP, D), f32),
        pltpu.SemaphoreType.DMA,
        pltpu.SemaphoreType.DMA,
        pltpu.SemaphoreType.DMA,
        pltpu.SemaphoreType.DMA,
        pltpu.SemaphoreType.DMA,
    ],
)
def _agg_kernel(g_hbm, src_hbm, dst_hbm, zeros_hbm, out_hbm,
                src_v, dst_v, rows0, rows1, rows2, rows3, acc,
                sem0, sem1, sem2, sem3, semi):
    cid = lax.axis_index("c")
    sid = lax.axis_index("s")
    wid = cid * NS + sid
    r0 = sid * RPT
    CH4 = CH // 4

    @pl.when(cid == 0)
    def _():
        pltpu.async_copy(g_hbm.at[pl.ds(r0, RPT)], acc.at[pl.ds(r0, RPT)], semi)

    @pl.when(cid != 0)
    def _():
        pltpu.async_copy(zeros_hbm.at[pl.ds(r0, RPT)], acc.at[pl.ds(r0, RPT)],
                         semi)

    pltpu.sync_copy(src_hbm.at[wid, pl.ds(0, CH4)], src_v)
    pltpu.sync_copy(dst_hbm.at[wid, pl.ds(0, CH4)], dst_v)
    pltpu.make_async_copy(
        zeros_hbm.at[pl.ds(r0, RPT)], acc.at[pl.ds(r0, RPT)], semi).wait()
    plsc.subcore_barrier()

    rows = (rows0, rows1, rows2, rows3)
    sems = (sem0, sem1, sem2, sem3)
    for q in range(4):
        if q:
            pltpu.sync_copy(src_hbm.at[wid, pl.ds(q * CH4, CH4)], src_v)
            pltpu.sync_copy(dst_hbm.at[wid, pl.ds(q * CH4, CH4)], dst_v)
        for j in range(3):
            pltpu.async_copy(g_hbm.at[src_v.at[j]], rows[j], sems[j])

        def body(i, carry):
            for j in range(4):
                c = 4 * i + j
                nc = c + 3
                jn = (j + 3) % 4

                @pl.when(nc < CH4)
                def _():
                    pltpu.async_copy(g_hbm.at[src_v.at[nc]], rows[jn], sems[jn])

                pltpu.make_async_copy(g_hbm.at[src_v.at[c]], rows[j],
                                      sems[j]).wait()
                pltpu.sync_copy(rows[j], acc.at[dst_v.at[c]], add=True)
            return carry

        lax.fori_loop(0, CH4 // 4, body, 0)
    plsc.subcore_barrier()
    pltpu.sync_copy(acc.at[pl.ds(r0, RPT)], out_hbm.at[cid, pl.ds(r0, RPT)])


def _dinv_body(degp_ref, dinv_ref):
    dv = degp_ref[...]
    deg = jnp.sum(dv, axis=0).astype(f32) + 1.0
    dinv_ref[...] = lax.rsqrt(deg)


_dinv = pl.pallas_call(_dinv_body, out_shape=jax.ShapeDtypeStruct((NPR, D), f32))


def _h_body(x_ref, w_ref, h_ref):
    h_ref[...] = jnp.dot(x_ref[...], w_ref[...], preferred_element_type=f32,
                         precision=lax.Precision.HIGHEST)


_h = pl.pallas_call(_h_body, out_shape=jax.ShapeDtypeStruct((NP, D), f32))


def _g_body(x_ref, w_ref, dinv_ref, g_ref):
    h = jnp.dot(x_ref[...], w_ref[...], preferred_element_type=f32,
                precision=lax.Precision.HIGHEST)
    g_ref[...] = dinv_ref[...] * h


_g = pl.pallas_call(_g_body, out_shape=jax.ShapeDtypeStruct((NP, D), f32))


def _out_body(agg_ref, dinv_ref, b_ref, o_ref):
    av = agg_ref[...]
    s = av[0, :N] + av[1, :N]
    o_ref[...] = jnp.maximum(dinv_ref[0:N] * s + b_ref[...], 0.0)


_out = pl.pallas_call(_out_body, out_shape=jax.ShapeDtypeStruct((N, D), f32))


def kernel(V, E, X, W, b):
    src = E[0].astype(i32)
    dst = E[1].astype(i32)
    pad = N + jnp.arange(E_PAD - EDG, dtype=i32) % (NP - N)
    src_p = jnp.concatenate([src, pad]).reshape(NW, CH, K)
    dst_p = jnp.concatenate([dst, pad]).reshape(NW, CH, K)
    zD = jnp.zeros((NP, D), f32)
    zH = jnp.zeros((NP,), i32)
    Xp = jnp.concatenate([X.astype(f32), jnp.zeros((NP - N, D), f32)], axis=0)
    degp = _deg_kernel(dst_p, zH).reshape(NW, NPR, D)
    dinv_col = _dinv(degp).reshape(NP, 1)
    g = _g(Xp, W.astype(f32), dinv_col)
    agg = _agg_kernel(g, src_p, dst_p, zD)
    return _out(agg, dinv_col, b.astype(f32).reshape(1, D))


# speedup vs baseline: 1.0127x; 1.0127x over previous
import functools

import jax
import jax.numpy as jnp
from jax import lax
from jax.experimental import pallas as pl
from jax.experimental.pallas import tpu as pltpu
from jax.experimental.pallas import tpu_sc as plsc

N = 10000
EDG = 320000
D = 128

NC, NS = 2, 16
NW = NC * NS
K = 64
EPT = (EDG + NW - 1) // NW
CH = 8 * ((EPT + 8 * K - 1) // (8 * K))
EPW = CH * K
E_PAD = EPW * NW
NP = N + 112
RPT = NP // NS
NPR = NP // D

f32 = jnp.float32
i32 = jnp.int32
_mesh = plsc.VectorSubcoreMesh(core_axis_name="c", subcore_axis_name="s")


@functools.partial(
    pl.kernel,
    out_type=jax.ShapeDtypeStruct((NW * NP,), i32),
    mesh=_mesh,
    compiler_params=pltpu.CompilerParams(needs_layout_passes=False),
    scratch_types=[
        pltpu.VMEM((CH, K), i32),
        pltpu.VMEM((NP,), i32),
    ],
)
def _deg_kernel(dst_hbm, zeros_hbm, out_hbm, dst_v, hist):
    cid = lax.axis_index("c")
    sid = lax.axis_index("s")
    wid = cid * NS + sid
    pltpu.sync_copy(dst_hbm.at[wid], dst_v)
    pltpu.sync_copy(zeros_hbm, hist)

    def body(c2, carry):
        for jj in range(2):
            for j in range(K // 16):
                dv = dst_v[2 * c2 + jj, pl.ds(j * 16, 16)]
                cnt, last = plsc.scan_count(dv)
                plsc.addupdate_scatter(hist, [dv], cnt, mask=last)
        return carry

    lax.fori_loop(0, CH // 2, body, 0)
    pltpu.sync_copy(hist, out_hbm.at[pl.ds(wid * NP, NP)])


@functools.partial(
    pl.kernel,
    out_type=jax.ShapeDtypeStruct((NC, NP, D), f32),
    mesh=_mesh,
    scratch_types=[
        pltpu.VMEM((CH // 4, K), i32),
        pltpu.VMEM((CH // 4, K), i32),
        pltpu.VMEM((K, D), f32),
        pltpu.VMEM((K, D), f32),
        pltpu.VMEM((K, D), f32),
        pltpu.VMEM((K, D), f32),
        pltpu.VMEM_SHARED((NP, D), f32),
        pltpu.SemaphoreType.DMA,
        pltpu.SemaphoreType.DMA,
        pltpu.SemaphoreType.DMA,
        pltpu.SemaphoreType.DMA,
        pltpu.SemaphoreType.DMA,
    ],
)
def _agg_kernel(g_hbm, src_hbm, dst_hbm, zeros_hbm, out_hbm,
                src_v, dst_v, rows0, rows1, rows2, rows3, acc,
                sem0, sem1, sem2, sem3, semi):
    cid = lax.axis_index("c")
    sid = lax.axis_index("s")
    wid = cid * NS + sid
    r0 = sid * RPT
    CH4 = CH // 4

    @pl.when(cid == 0)
    def _():
        pltpu.async_copy(g_hbm.at[pl.ds(r0, RPT)], acc.at[pl.ds(r0, RPT)], semi)

    @pl.when(cid != 0)
    def _():
        pltpu.async_copy(zeros_hbm.at[pl.ds(r0, RPT)], acc.at[pl.ds(r0, RPT)],
                         semi)

    pltpu.sync_copy(src_hbm.at[wid, pl.ds(0, CH4)], src_v)
    pltpu.sync_copy(dst_hbm.at[wid, pl.ds(0, CH4)], dst_v)
    pltpu.make_async_copy(
        zeros_hbm.at[pl.ds(r0, RPT)], acc.at[pl.ds(r0, RPT)], semi).wait()
    plsc.subcore_barrier()

    rows = (rows0, rows1, rows2, rows3)
    sems = (sem0, sem1, sem2, sem3)
    for q in range(4):
        if q:
            pltpu.sync_copy(src_hbm.at[wid, pl.ds(q * CH4, CH4)], src_v)
            pltpu.sync_copy(dst_hbm.at[wid, pl.ds(q * CH4, CH4)], dst_v)
        for j in range(3):
            pltpu.async_copy(g_hbm.at[src_v.at[j]], rows[j], sems[j])

        def body(i, carry):
            for j in range(4):
                c = 4 * i + j
                nc = c + 3
                jn = (j + 3) % 4

                @pl.when(nc < CH4)
                def _():
                    pltpu.async_copy(g_hbm.at[src_v.at[nc]], rows[jn], sems[jn])

                pltpu.make_async_copy(g_hbm.at[src_v.at[c]], rows[j],
                                      sems[j]).wait()
                pltpu.sync_copy(rows[j], acc.at[dst_v.at[c]], add=True)
            return carry

        lax.fori_loop(0, CH4 // 4, body, 0)
    plsc.subcore_barrier()
    pltpu.sync_copy(acc.at[pl.ds(r0, RPT)], out_hbm.at[cid, pl.ds(r0, RPT)])


def _dinv_body(degp_ref, dinv_ref):
    dv = degp_ref[...]
    deg = jnp.sum(dv, axis=0).astype(f32) + 1.0
    dinv_ref[...] = lax.rsqrt(deg)


_dinv = pl.pallas_call(_dinv_body, out_shape=jax.ShapeDtypeStruct((NPR, D), f32))


def _h_body(x_ref, w_ref, h_ref):
    h_ref[...] = jnp.dot(x_ref[...], w_ref[...], preferred_element_type=f32,
                         precision=lax.Precision.HIGHEST)


_h = pl.pallas_call(_h_body, out_shape=jax.ShapeDtypeStruct((NP, D), f32))


def _g_body(h_ref, dinv_ref, g_ref):
    g_ref[...] = dinv_ref[...] * h_ref[...]


_g = pl.pallas_call(_g_body, out_shape=jax.ShapeDtypeStruct((NP, D), f32))


def _out_body(agg_ref, dinv_ref, b_ref, o_ref):
    av = agg_ref[...]
    s = av[0, :N] + av[1, :N]
    o_ref[...] = jnp.maximum(dinv_ref[0:N] * s + b_ref[...], 0.0)


_out = pl.pallas_call(_out_body, out_shape=jax.ShapeDtypeStruct((N, D), f32))


def kernel(V, E, X, W, b):
    src = E[0].astype(i32)
    dst = E[1].astype(i32)
    pad = N + jnp.arange(E_PAD - EDG, dtype=i32) % (NP - N)
    src_p = jnp.concatenate([src, pad]).reshape(NW, CH, K)
    dst_p = jnp.concatenate([dst, pad]).reshape(NW, CH, K)
    zD = jnp.zeros((NP, D), f32)
    zH = jnp.zeros((NP,), i32)
    Xp = jnp.concatenate([X.astype(f32), jnp.zeros((NP - N, D), f32)], axis=0)
    degp = _deg_kernel(dst_p, zH).reshape(NW, NPR, D)
    h = _h(Xp, W.astype(f32))
    dinv_col = _dinv(degp).reshape(NP, 1)
    g = _g(h, dinv_col)
    agg = _agg_kernel(g, src_p, dst_p, zD)
    return _out(agg, dinv_col, b.astype(f32).reshape(1, D))
